# baseline (speedup 1.0000x reference)
import functools

import jax
import jax.numpy as jnp
from jax import lax
from jax.experimental import pallas as pl
from jax.experimental.pallas import tpu as pltpu
from jax.experimental.pallas import tpu_sc as plsc

_NUM_CORES = 2
_NUM_SUBCORES = 16
_NUM_WORKERS = _NUM_CORES * _NUM_SUBCORES
_LANES = 16
_IDX_CHUNK = 128


def _bt_sc_call(idx_a, idx_b, table2, w):
    n_chunks_total, chunk = idx_a.shape
    batch = n_chunks_total * chunk
    width = table2.shape[1]
    dim = width // 2
    b_per_w = batch // _NUM_WORKERS
    chunks_per_w = b_per_w // chunk
    groups_per_w = b_per_w // _LANES
    groups_per_chunk = chunk // _LANES

    mesh = plsc.VectorSubcoreMesh(core_axis_name="c", subcore_axis_name="s")

    @functools.partial(
        pl.kernel,
        mesh=mesh,
        out_type=jax.ShapeDtypeStruct((batch,), jnp.float32),
        compiler_params=pltpu.CompilerParams(needs_layout_passes=False),
        scratch_types=[
            pltpu.VMEM((chunks_per_w, chunk), jnp.int32),
            pltpu.VMEM((chunks_per_w, chunk), jnp.int32),
            pltpu.VMEM((chunks_per_w, chunk), jnp.int32),
            pltpu.VMEM((chunks_per_w, chunk), jnp.int32),
            pltpu.VMEM((dim,), jnp.float32),
            pltpu.VMEM((b_per_w // 2, width), jnp.float32),
            pltpu.VMEM((b_per_w // 2, width), jnp.float32),
            pltpu.VMEM((b_per_w,), jnp.float32),
            pltpu.SemaphoreType.DMA,
        ],
    )
    def run(idx_a_hbm, idx_b_hbm, table_hbm, w_hbm, out_hbm,
            idxa_v, idxb_v, rowa_v, rowb_v, w_v, rows_a, rows_b, out_v, sem):
        wid = lax.axis_index("s") * _NUM_CORES + lax.axis_index("c")
        crow = wid * chunks_per_w
        pltpu.sync_copy(idx_a_hbm.at[pl.ds(crow, chunks_per_w)], idxa_v)
        pltpu.sync_copy(idx_b_hbm.at[pl.ds(crow, chunks_per_w)], idxb_v)
        pltpu.sync_copy(w_hbm, w_v)

        for r in range(chunks_per_w):
            for c in range(chunk // _LANES):
                s = pl.ds(c * _LANES, _LANES)
                rowa_v[r, s] = idxa_v[r, s] >> 1
                rowb_v[r, s] = idxb_v[r, s] >> 1

        w_vecs = [w_v[pl.ds(k * _LANES, _LANES)] for k in range(dim // _LANES)]
        w_scal = [w_vecs[d // _LANES][d % _LANES] for d in range(dim)]

        for half in range(2):
            copies = []
            for j in range(chunks_per_w // 2):
                jj = half * (chunks_per_w // 2) + j
                copies.append(pltpu.async_copy(
                    table_hbm.at[rowa_v.at[jj]],
                    rows_a.at[pl.ds(j * chunk, chunk)], sem))
                copies.append(pltpu.async_copy(
                    table_hbm.at[rowb_v.at[jj]],
                    rows_b.at[pl.ds(j * chunk, chunk)], sem))
            for c in copies:
                c.wait()

            def group_body(g, carry, half=half):
                goff = half * (groups_per_w // 2) + g
                ia = idxa_v[goff // groups_per_chunk,
                            pl.ds((goff % groups_per_chunk) * _LANES, _LANES)]
                ib = idxb_v[goff // groups_per_chunk,
                            pl.ds((goff % groups_per_chunk) * _LANES, _LANES)]
                rid = g * _LANES + lax.iota(jnp.int32, _LANES)
                cola = (ia & 1) * dim
                colb = (ib & 1) * dim
                acc = jnp.zeros((_LANES,), jnp.float32)
                one = jnp.ones((_LANES,), jnp.int32)
                for d in range(dim):
                    va = plsc.load_gather(rows_a, [rid, cola])
                    vb = plsc.load_gather(rows_b, [rid, colb])
                    acc = acc + (va - vb) * w_scal[d]
                    cola = cola + one
                    colb = colb + one
                res = 1.0 / (1.0 + jnp.exp(-acc))
                out_v[pl.ds(pl.multiple_of(goff * _LANES, _LANES), _LANES)] = res
                return carry

            lax.fori_loop(0, groups_per_w // 2, group_body, 0)

        pltpu.sync_copy(out_v, out_hbm.at[pl.ds(wid * b_per_w, b_per_w)])

    return run(idx_a, idx_b, table2, w)


def kernel(item_a, item_b, item_strengths, head_w, head_b):
    batch = item_a.shape[0]
    vocab, dim = item_strengths.shape
    idx_a = item_a.astype(jnp.int32).reshape(batch // _IDX_CHUNK, _IDX_CHUNK)
    idx_b = item_b.astype(jnp.int32).reshape(batch // _IDX_CHUNK, _IDX_CHUNK)
    table2 = item_strengths.reshape(vocab // 2, 2 * dim)
    w = head_w.reshape(dim).astype(jnp.float32)
    out = _bt_sc_call(idx_a, idx_b, table2, w)
    return out.reshape(batch, 1)


# speedup vs baseline: 1.7930x; 1.7930x over previous
import functools

import jax
import jax.numpy as jnp
from jax import lax
from jax.experimental import pallas as pl
from jax.experimental.pallas import tpu as pltpu
from jax.experimental.pallas import tpu_sc as plsc

_NUM_CORES = 2
_NUM_SUBCORES = 16
_NUM_WORKERS = _NUM_CORES * _NUM_SUBCORES
_LANES = 16
_IDX_CHUNK = 128


def _bt_sc_call(idx_a, idx_b, table, w):
    n_chunks_total, chunk = idx_a.shape
    batch = n_chunks_total * chunk
    dim = table.shape[1]
    b_per_w = batch // _NUM_WORKERS
    chunks_per_w = b_per_w // chunk
    groups_per_w = b_per_w // _LANES
    groups_per_chunk = chunk // _LANES

    mesh = plsc.VectorSubcoreMesh(core_axis_name="c", subcore_axis_name="s")

    @functools.partial(
        pl.kernel,
        mesh=mesh,
        out_type=jax.ShapeDtypeStruct((batch,), jnp.float32),
        compiler_params=pltpu.CompilerParams(needs_layout_passes=False),
        scratch_types=[
            pltpu.VMEM((chunks_per_w, chunk), jnp.int32),
            pltpu.VMEM((chunks_per_w, chunk), jnp.int32),
            pltpu.VMEM((2 * dim,), jnp.float32),
            pltpu.VMEM((b_per_w, dim), jnp.float32),
            pltpu.VMEM((b_per_w,), jnp.float32),
            pltpu.VMEM((b_per_w,), jnp.float32),
            pltpu.SemaphoreType.DMA,
        ],
    )
    def run(idx_a_hbm, idx_b_hbm, table_hbm, w_hbm, out_hbm,
            idxa_v, idxb_v, w_v, rows, acc_v, out_v, sem):
        wid = lax.axis_index("s") * _NUM_CORES + lax.axis_index("c")
        crow = wid * chunks_per_w
        pltpu.sync_copy(idx_a_hbm.at[pl.ds(crow, chunks_per_w)], idxa_v)
        pltpu.sync_copy(idx_b_hbm.at[pl.ds(crow, chunks_per_w)], idxb_v)
        pltpu.sync_copy(w_hbm, w_v.at[pl.ds(0, dim)])
        pltpu.sync_copy(w_hbm, w_v.at[pl.ds(dim, dim)])

        for idx_v in (idxa_v, idxb_v):
            is_b = idx_v is idxb_v

            def issue_body(g, carry, idx_v=idx_v):
                iv = idx_v[g // groups_per_chunk,
                           pl.ds((g % groups_per_chunk) * _LANES, _LANES)]
                for l in range(_LANES):
                    pltpu.async_copy(
                        table_hbm.at[pl.ds(iv[l], 1), :],
                        rows.at[pl.ds(g * _LANES + l, 1), :], sem)
                return carry

            lax.fori_loop(0, groups_per_w, issue_body, 0)
            pltpu.make_async_copy(
                table_hbm.at[pl.ds(0, b_per_w), :], rows, sem).wait()

            def group_body(g, carry, is_b=is_b):
                rid = g * _LANES + lax.iota(jnp.int32, _LANES)
                dpl = lax.iota(jnp.int32, _LANES)
                one = jnp.ones((_LANES,), jnp.int32)
                msk = jnp.full((_LANES,), dim - 1, jnp.int32)
                acc = jnp.zeros((_LANES,), jnp.float32)
                for d in range(dim):
                    v = plsc.load_gather(rows, [rid, dpl])
                    wd = w_v[pl.ds(d, _LANES)]
                    acc = acc + v * wd
                    dpl = (dpl + one) & msk
                s = pl.ds(pl.multiple_of(g * _LANES, _LANES), _LANES)
                if not is_b:
                    acc_v[s] = acc
                else:
                    out_v[s] = 1.0 / (1.0 + jnp.exp(acc - acc_v[s]))
                return carry

            lax.fori_loop(0, groups_per_w, group_body, 0)

        pltpu.sync_copy(out_v, out_hbm.at[pl.ds(wid * b_per_w, b_per_w)])

    return run(idx_a, idx_b, table, w)


def kernel(item_a, item_b, item_strengths, head_w, head_b):
    batch = item_a.shape[0]
    dim = item_strengths.shape[1]
    idx_a = item_a.astype(jnp.int32).reshape(batch // _IDX_CHUNK, _IDX_CHUNK)
    idx_b = item_b.astype(jnp.int32).reshape(batch // _IDX_CHUNK, _IDX_CHUNK)
    w = head_w.reshape(dim).astype(jnp.float32)
    out = _bt_sc_call(idx_a, idx_b, item_strengths, w)
    return out.reshape(batch, 1)
